# baseline (speedup 1.0000x reference)
import jax
import jax.numpy as jnp
from jax.experimental import pallas as pl
from jax.experimental.pallas import tpu as pltpu

_BI = 200


def _body(x_ref, w0t_ref, b0_ref, w1t_ref, b1_ref, a0_ref, a1_ref,
          out_ref, acc_ref):
    i = pl.program_id(0)
    ni = pl.num_programs(0)
    xb = x_ref[pl.ds(i * _BI, _BI), :]
    dnw = (((1,), (1,)), ((), ()))
    y0 = (jax.lax.dot_general(xb, w0t_ref[...], dnw,
                              preferred_element_type=jnp.float32)
          + b0_ref[...])
    y1 = (jax.lax.dot_general(xb, w1t_ref[...], dnw,
                              preferred_element_type=jnp.float32)
          + b1_ref[...])
    y0t = y0.T.astype(jnp.bfloat16)
    y1t = y1.T.astype(jnp.bfloat16)
    a0 = a0_ref[...].astype(jnp.bfloat16)
    a1 = a1_ref[...].astype(jnp.bfloat16)
    dn = (((1,), (0,)), ((), ()))
    p0 = jax.lax.dot_general(y0t, a0, dn, preferred_element_type=jnp.float32)
    p1 = jax.lax.dot_general(y1t, a1, dn, preferred_element_type=jnp.float32)

    @pl.when(i == 0)
    def _init():
        acc_ref[...] = p0 + p1

    @pl.when(i > 0)
    def _acc():
        acc_ref[...] += p0 + p1

    @pl.when(i == ni - 1)
    def _finish():
        out_ref[...] = acc_ref[...].T


def kernel(A_r0, A_r1, x, W0, b0, W1, b1):
    n, d_in = x.shape
    d_out = W0.shape[0]
    return pl.pallas_call(
        _body,
        grid=(n // _BI,),
        in_specs=[
            pl.BlockSpec((n, d_in), lambda i: (0, 0)),
            pl.BlockSpec((d_out, d_in), lambda i: (0, 0)),
            pl.BlockSpec((1, d_out), lambda i: (0, 0)),
            pl.BlockSpec((d_out, d_in), lambda i: (0, 0)),
            pl.BlockSpec((1, d_out), lambda i: (0, 0)),
            pl.BlockSpec((_BI, n), lambda i: (i, 0)),
            pl.BlockSpec((_BI, n), lambda i: (i, 0)),
        ],
        out_specs=pl.BlockSpec((n, d_out), lambda i: (0, 0)),
        out_shape=jax.ShapeDtypeStruct((n, d_out), x.dtype),
        scratch_shapes=[pltpu.VMEM((d_out, n), jnp.float32)],
    )(x, W0, b0[None, :], W1, b1[None, :], A_r0, A_r1)


# speedup vs baseline: 1.0005x; 1.0005x over previous
import jax
import jax.numpy as jnp
from jax.experimental import pallas as pl
from jax.experimental.pallas import tpu as pltpu

_BI = 200


def _body(x_ref, w0_ref, b0_ref, w1_ref, b1_ref, a0_ref, a1_ref,
          out_ref, acc_ref, a01_ref):
    i = pl.program_id(0)
    ni = pl.num_programs(0)
    xb = x_ref[...]
    dnw = (((1,), (1,)), ((), ()))
    y0 = (jax.lax.dot_general(xb, w0_ref[...], dnw,
                              preferred_element_type=jnp.float32)
          + b0_ref[...])
    y1 = (jax.lax.dot_general(xb, w1_ref[...], dnw,
                              preferred_element_type=jnp.float32)
          + b1_ref[...])
    y01t = jnp.concatenate([y0.T, y1.T], axis=1).astype(jnp.bfloat16)
    a01_ref[0:_BI, :] = a0_ref[...].astype(jnp.bfloat16)
    a01_ref[_BI:2 * _BI, :] = a1_ref[...].astype(jnp.bfloat16)
    dn = (((1,), (0,)), ((), ()))
    p = jax.lax.dot_general(y01t, a01_ref[...], dn,
                            preferred_element_type=jnp.float32)

    @pl.when(i == 0)
    def _init():
        acc_ref[...] = p

    @pl.when(i > 0)
    def _acc():
        acc_ref[...] += p

    @pl.when(i == ni - 1)
    def _finish():
        out_ref[...] = acc_ref[...].T


def kernel(A_r0, A_r1, x, W0, b0, W1, b1):
    n, d_in = x.shape
    d_out = W0.shape[0]
    return pl.pallas_call(
        _body,
        grid=(n // _BI,),
        in_specs=[
            pl.BlockSpec((_BI, d_in), lambda i: (i, 0)),
            pl.BlockSpec((d_out, d_in), lambda i: (0, 0)),
            pl.BlockSpec((1, d_out), lambda i: (0, 0)),
            pl.BlockSpec((d_out, d_in), lambda i: (0, 0)),
            pl.BlockSpec((1, d_out), lambda i: (0, 0)),
            pl.BlockSpec((_BI, n), lambda i: (i, 0)),
            pl.BlockSpec((_BI, n), lambda i: (i, 0)),
        ],
        out_specs=pl.BlockSpec((n, d_out), lambda i: (0, 0)),
        out_shape=jax.ShapeDtypeStruct((n, d_out), x.dtype),
        scratch_shapes=[
            pltpu.VMEM((d_out, n), jnp.float32),
            pltpu.VMEM((2 * _BI, n), jnp.bfloat16),
        ],
    )(x, W0, b0[None, :], W1, b1[None, :], A_r0, A_r1)


# speedup vs baseline: 1.0073x; 1.0068x over previous
import jax
import jax.numpy as jnp
from jax.experimental import pallas as pl
from jax.experimental.pallas import tpu as pltpu

_BI = 200


def _body(x_ref, w0t_ref, b0_ref, w1t_ref, b1_ref, a0_ref, a1_ref,
          out_ref, acc_ref, p0_ref, p1_ref):
    i = pl.program_id(0)
    ni = pl.num_programs(0)
    xb = x_ref[...]
    dnw = (((1,), (1,)), ((), ()))
    y0 = (jax.lax.dot_general(xb, w0t_ref[...], dnw,
                              preferred_element_type=jnp.float32)
          + b0_ref[...])
    y1 = (jax.lax.dot_general(xb, w1t_ref[...], dnw,
                              preferred_element_type=jnp.float32)
          + b1_ref[...])
    @pl.when(i == 1)
    def _acc_init():
        acc_ref[...] = p0_ref[...] + p1_ref[...]

    @pl.when(i > 1)
    def _acc_fold():
        acc_ref[...] += p0_ref[...] + p1_ref[...]

    y0t = y0.T.astype(jnp.bfloat16)
    y1t = y1.T.astype(jnp.bfloat16)
    a0 = a0_ref[...].astype(jnp.bfloat16)
    a1 = a1_ref[...].astype(jnp.bfloat16)
    dn = (((1,), (0,)), ((), ()))
    p0 = jax.lax.dot_general(y0t, a0, dn, preferred_element_type=jnp.float32)
    p1 = jax.lax.dot_general(y1t, a1, dn, preferred_element_type=jnp.float32)
    p0_ref[...] = p0
    p1_ref[...] = p1

    @pl.when(i == ni - 1)
    def _finish():
        out_ref[...] = (acc_ref[...] + p0 + p1).T


def kernel(A_r0, A_r1, x, W0, b0, W1, b1):
    n, d_in = x.shape
    d_out = W0.shape[0]
    return pl.pallas_call(
        _body,
        grid=(n // _BI,),
        in_specs=[
            pl.BlockSpec((_BI, d_in), lambda i: (i, 0)),
            pl.BlockSpec((d_out, d_in), lambda i: (0, 0)),
            pl.BlockSpec((1, d_out), lambda i: (0, 0)),
            pl.BlockSpec((d_out, d_in), lambda i: (0, 0)),
            pl.BlockSpec((1, d_out), lambda i: (0, 0)),
            pl.BlockSpec((_BI, n), lambda i: (i, 0)),
            pl.BlockSpec((_BI, n), lambda i: (i, 0)),
        ],
        out_specs=pl.BlockSpec((n, d_out), lambda i: (0, 0)),
        out_shape=jax.ShapeDtypeStruct((n, d_out), x.dtype),
        scratch_shapes=[pltpu.VMEM((d_out, n), jnp.float32),
                        pltpu.VMEM((d_out, n), jnp.float32),
                        pltpu.VMEM((d_out, n), jnp.float32)],
        compiler_params=pltpu.CompilerParams(vmem_limit_bytes=64 * 1024 * 1024),
    )(x, W0, b0[None, :], W1, b1[None, :], A_r0, A_r1)
